# baseline (speedup 1.0000x reference)
import jax
import jax.numpy as jnp
from jax.experimental import pallas as pl
from jax.experimental.pallas import tpu as pltpu

_BATCH, _STEPS, _NODES = 64, 24, 64
_D, _QDIM, _PLM = 32, 5, 768
_ALPHA = 0.5
_BB = 8
_NP = _BB // 2
_GRID = _BATCH // _BB
_FP = 72
_H2 = 2 * _D
_L = 2 * _NODES
_PREC = jax.lax.Precision.DEFAULT


def _rnn_body(obsT_ref, mask_ref, maskT_ref, avg_ref, avgT_ref, len_ref,
              vprT_ref, rWT_ref, Wf1T_ref, bf1_ref, Wf2T_ref, bf2_ref,
              Wg1T_ref, bg1_ref, Wg2T_ref, bg2_ref,
              WruT_ref, WcT_ref, bruT_ref, bcT_ref,
              out_ref,
              adjE2_s, adjW2_s, qv5_s, bbru_s, bbc_s, rrow_s):

    @pl.when(pl.program_id(0) == 0)
    def _prologue():
        vprT = vprT_ref[...]
        qhT = jnp.maximum(
            jax.lax.dot(Wf1T_ref[...], vprT, precision=_PREC) + bf1_ref[...],
            0.0)
        qvT = jax.lax.dot(Wf2T_ref[...], qhT, precision=_PREC) + bf2_ref[...]
        ghT = jnp.maximum(
            jax.lax.dot(Wg1T_ref[...], vprT, precision=_PREC) + bg1_ref[...],
            0.0)
        neT = jax.lax.dot(Wg2T_ref[...], ghT, precision=_PREC) + bg2_ref[...]
        nrm = jnp.sqrt(jnp.sum(neT * neT, axis=0, keepdims=True))
        neT = neT / jnp.maximum(nrm, 1e-12)
        logits = jax.lax.dot_general(neT, neT, (((0,), (0,)), ((), ())),
                                     precision=_PREC)
        mx = jnp.max(logits, axis=0, keepdims=True)
        e = jnp.exp(logits - mx)
        adjT = e / jnp.sum(e, axis=0, keepdims=True)
        eye = (jax.lax.broadcasted_iota(jnp.int32, (_NODES, _NODES), 0) ==
               jax.lax.broadcasted_iota(jnp.int32, (_NODES, _NODES), 1)
               ).astype(jnp.float32)
        adjET = adjT * (1.0 - eye)
        adjWT = adjET * rWT_ref[...]
        lane2 = lambda a: jnp.concatenate([a, a], axis=1)
        adjE2_s[...] = jnp.concatenate([lane2(adjET)] * 2, axis=0)
        adjW2_s[...] = jnp.concatenate([lane2(adjWT)] * 2, axis=0)
        qvT2 = lane2(qvT)
        qv5_s[...] = jnp.concatenate(
            [jnp.broadcast_to(qvT2[d:d + 1, :], (_FP, _L))
             for d in range(_QDIM)], axis=0)
        bbru_s[...] = lane2(jax.lax.dot(bruT_ref[...], qvT, precision=_PREC))
        bbc_s[...] = lane2(jax.lax.dot(bcT_ref[...], qvT, precision=_PREC))

    vto2 = jnp.sum(mask_ref[...], axis=1)
    vtoT = jnp.sum(maskT_ref[0], axis=0)
    rrow_s[...] = _ALPHA * jnp.tanh(avgT_ref[0] / (vtoT[None] + 1.0))
    lb3 = len_ref[0][:, None, :]
    bmask = (jax.lax.broadcasted_iota(jnp.int32, (_L, _L), 0) // _NODES ==
             jax.lax.broadcasted_iota(jnp.int32, (_L, _L), 1) // _NODES
             ).astype(jnp.float32)
    zpad = jnp.zeros((_NP, _FP - 2 * _D - 1, _L), jnp.float32)
    adjE2 = adjE2_s[...]
    adjW2 = adjW2_s[...]
    qv5 = qv5_s[...]
    bbru = bbru_s[...]
    bbc = bbc_s[...]
    WruT = WruT_ref[...]
    WcT = WcT_ref[...]

    def step_fn(step, carry):
        hT, outT = carry
        m3 = mask_ref[:, step, :][:, None, :]
        rar = _ALPHA * jnp.tanh(avg_ref[:, step, :] / (vto2 + 1.0))
        rar3 = rar[:, None, :]
        rT = rrow_s[step]
        rar_rows = jnp.stack(
            [jnp.concatenate([rT[:, 2 * p:2 * p + 1],
                              rT[:, 2 * p + 1:2 * p + 2]], axis=0)
             for p in range(_NP)], axis=0)
        dr = jnp.abs(rar_rows - rar3)
        Mm = (adjE2[None] - adjW2[None] * dr) * bmask[None]
        obsT = obsT_ref[:, step]
        rz = jnp.concatenate([rar3, zpad], axis=1)
        xhT = jnp.concatenate([obsT, hT, rz], axis=1)
        xhmT = m3 * xhT
        combT = m3 * jnp.stack(
            [jax.lax.dot(xhmT[p], Mm[p], precision=_PREC)
             for p in range(_NP)], axis=0) + xhT
        accT = jnp.stack(
            [jax.lax.dot(
                WruT,
                jnp.concatenate([combT[p]] * _QDIM, axis=0) * qv5,
                precision=_PREC) for p in range(_NP)], axis=0) + bbru[None]
        r = jax.nn.sigmoid(accT[:, :_D])
        u = jax.nn.sigmoid(accT[:, _D:_H2])
        mgt = m3 > 0.0
        h_rT = jnp.where(mgt, r * hT, hT)
        xcT = jnp.concatenate([obsT, h_rT, rz], axis=1)
        candT = jnp.tanh(jnp.stack(
            [jax.lax.dot(
                WcT,
                jnp.concatenate([xcT[p]] * _QDIM, axis=0) * qv5,
                precision=_PREC) for p in range(_NP)], axis=0) + bbc[None])
        h_new = jnp.where(mgt, (1.0 - u) * h_rT + u * candT, hT)
        out_new = jnp.where(lb3 == step + 1, h_new, outT)
        return h_new, out_new

    h0 = jnp.zeros((_NP, _D, _L), jnp.float32)
    _, outT = jax.lax.fori_loop(0, _STEPS, step_fn, (h0, h0))
    out_ref[...] = outT


def _pack2(a):
    return (a.reshape(_BATCH // 2, 2, _STEPS, _NODES)
            .transpose(0, 2, 1, 3).reshape(_BATCH // 2, _STEPS, _L))


def _wflat(w):
    wp = jnp.concatenate(
        [w[:, :_D], w[:, _D + 1:], w[:, _D:_D + 1],
         jnp.zeros((_QDIM, _FP - 2 * _D - 1, w.shape[2]), w.dtype)], axis=1)
    return wp.reshape(_QDIM * _FP, w.shape[2]).T


def kernel(obs_emb, observed_mask, lengths, avg_interval, var_plm_rep,
           rarity_W, Wf1, bf1, Wf2, bf2, Wg1, bg1, Wg2, bg2,
           Wu, bu, Wr, br, Wc, bc):
    obsT2 = (obs_emb.transpose(0, 1, 3, 2)
             .reshape(_BATCH // 2, 2, _STEPS, _D, _NODES)
             .transpose(0, 2, 3, 1, 4)
             .reshape(_BATCH // 2, _STEPS, _D, _L))
    mask2 = _pack2(observed_mask)
    avg2 = _pack2(avg_interval)
    len2 = (jnp.broadcast_to(lengths, (_BATCH, _NODES))
            .reshape(_GRID, _NP, _L))
    maskT = (observed_mask.transpose(1, 2, 0)
             .reshape(_STEPS, _NODES, _GRID, _BB)
             .transpose(2, 0, 1, 3))
    avgT = (avg_interval.transpose(1, 2, 0)
            .reshape(_STEPS, _NODES, _GRID, _BB)
            .transpose(2, 0, 1, 3))
    WruT = _wflat(jnp.stack([Wr, Wu], axis=2).reshape(_QDIM, 2 * _D + 1,
                                                      2 * _D))
    WcT = _wflat(Wc)
    bruT = jnp.concatenate([br, bu], axis=1).T
    bcT = bc.T

    full = lambda nd: (lambda i: (0,) * nd)
    outT = pl.pallas_call(
        _rnn_body,
        grid=(_GRID,),
        in_specs=[
            pl.BlockSpec((_NP, _STEPS, _D, _L), lambda i: (i, 0, 0, 0)),
            pl.BlockSpec((_NP, _STEPS, _L), lambda i: (i, 0, 0)),
            pl.BlockSpec((1, _STEPS, _NODES, _BB), lambda i: (i, 0, 0, 0)),
            pl.BlockSpec((_NP, _STEPS, _L), lambda i: (i, 0, 0)),
            pl.BlockSpec((1, _STEPS, _NODES, _BB), lambda i: (i, 0, 0, 0)),
            pl.BlockSpec((1, _NP, _L), lambda i: (i, 0, 0)),
            pl.BlockSpec((_PLM, _NODES), full(2)),
            pl.BlockSpec((_NODES, _NODES), full(2)),
            pl.BlockSpec((_H2, _PLM), full(2)),
            pl.BlockSpec((_H2, 1), full(2)),
            pl.BlockSpec((_QDIM, _H2), full(2)),
            pl.BlockSpec((_QDIM, 1), full(2)),
            pl.BlockSpec((_H2, _PLM), full(2)),
            pl.BlockSpec((_H2, 1), full(2)),
            pl.BlockSpec((8, _H2), full(2)),
            pl.BlockSpec((8, 1), full(2)),
            pl.BlockSpec((2 * _D, _QDIM * _FP), full(2)),
            pl.BlockSpec((_D, _QDIM * _FP), full(2)),
            pl.BlockSpec((2 * _D, _QDIM), full(2)),
            pl.BlockSpec((_D, _QDIM), full(2)),
        ],
        out_specs=pl.BlockSpec((_NP, _D, _L), lambda i: (i, 0, 0)),
        out_shape=jax.ShapeDtypeStruct((_BATCH // 2, _D, _L), jnp.float32),
        scratch_shapes=[
            pltpu.VMEM((_L, _L), jnp.float32),
            pltpu.VMEM((_L, _L), jnp.float32),
            pltpu.VMEM((_QDIM * _FP, _L), jnp.float32),
            pltpu.VMEM((2 * _D, _L), jnp.float32),
            pltpu.VMEM((_D, _L), jnp.float32),
            pltpu.VMEM((_STEPS, _NODES, _BB), jnp.float32),
        ],
        compiler_params=pltpu.CompilerParams(
            dimension_semantics=("arbitrary",)),
    )(obsT2, mask2, maskT, avg2, avgT, len2.astype(jnp.int32),
      var_plm_rep.T, rarity_W.T, Wf1.T, bf1.reshape(-1, 1),
      Wf2.T, bf2.reshape(-1, 1), Wg1.T, bg1.reshape(-1, 1),
      Wg2.T, bg2.reshape(-1, 1), WruT, WcT, bruT, bcT)
    return (outT.reshape(_BATCH // 2, _D, 2, _NODES)
            .transpose(0, 2, 3, 1).reshape(_BATCH, _NODES, _D))


# speedup vs baseline: 1.2712x; 1.2712x over previous
import jax
import jax.numpy as jnp
from jax.experimental import pallas as pl
from jax.experimental.pallas import tpu as pltpu

_BATCH, _STEPS, _NODES = 64, 24, 64
_D, _QDIM, _PLM = 32, 5, 768
_ALPHA = 0.5
_BB = 8
_NF = 2 * _D + 1
_FP = 72
_H2 = 2 * _D
_PREC = jax.lax.Precision.DEFAULT


def _rnn_body(obsT_ref, mask_ref, maskT_ref, avg_ref, avgT_ref, len_ref,
              vprT_ref, rWT_ref, Wf1T_ref, bf1_ref, Wf2T_ref, bf2_ref,
              Wg1T_ref, bg1_ref, Wg2T_ref, bg2_ref,
              WruT_ref, WcT_ref, bruT_ref, bcT_ref,
              out_ref,
              adjET_s, adjWT_s, qv5_s, bbru_s, bbc_s, rrow_s):

    @pl.when(pl.program_id(0) == 0)
    def _prologue():
        vprT = vprT_ref[...]
        qhT = jnp.maximum(
            jax.lax.dot(Wf1T_ref[...], vprT, precision=_PREC) + bf1_ref[...],
            0.0)
        qvT = jax.lax.dot(Wf2T_ref[...], qhT, precision=_PREC) + bf2_ref[...]
        ghT = jnp.maximum(
            jax.lax.dot(Wg1T_ref[...], vprT, precision=_PREC) + bg1_ref[...],
            0.0)
        neT = jax.lax.dot(Wg2T_ref[...], ghT, precision=_PREC) + bg2_ref[...]
        nrm = jnp.sqrt(jnp.sum(neT * neT, axis=0, keepdims=True))
        neT = neT / jnp.maximum(nrm, 1e-12)
        logits = jax.lax.dot_general(neT, neT, (((0,), (0,)), ((), ())),
                                     precision=_PREC)
        mx = jnp.max(logits, axis=0, keepdims=True)
        e = jnp.exp(logits - mx)
        adjT = e / jnp.sum(e, axis=0, keepdims=True)
        eye = (jax.lax.broadcasted_iota(jnp.int32, (_NODES, _NODES), 0) ==
               jax.lax.broadcasted_iota(jnp.int32, (_NODES, _NODES), 1)
               ).astype(jnp.float32)
        adjET = adjT * (1.0 - eye)
        adjET_s[...] = adjET
        adjWT_s[...] = adjET * rWT_ref[...]
        qv5_s[...] = jnp.concatenate(
            [jnp.broadcast_to(qvT[d:d + 1, :], (_FP, _NODES))
             for d in range(_QDIM)], axis=0)
        bbru_s[...] = jax.lax.dot(bruT_ref[...], qvT, precision=_PREC)
        bbc_s[...] = jax.lax.dot(bcT_ref[...], qvT, precision=_PREC)

    vto = jnp.sum(mask_ref[...], axis=1)
    vtoT = jnp.sum(maskT_ref[0], axis=0)
    rrow_s[...] = _ALPHA * jnp.tanh(avgT_ref[0] / (vtoT[None] + 1.0))
    lb3 = len_ref[...].reshape(_BB, 1, 1)
    zpad = jnp.zeros((_BB, _FP - _NF, _NODES), jnp.float32)
    adjET = adjET_s[...]
    adjWT = adjWT_s[...]
    qv5 = qv5_s[...]
    bbru = bbru_s[...]
    bbc = bbc_s[...]
    WruT = WruT_ref[...]
    WcT = WcT_ref[...]

    def step_fn(step, carry):
        hT, outT = carry
        m3 = mask_ref[:, step, :][:, None, :]
        rar = _ALPHA * jnp.tanh(avg_ref[:, step, :] / (vto + 1.0))
        rar3 = rar[:, None, :]
        rT = rrow_s[step]
        rar_rows = jnp.stack([rT[:, b:b + 1] for b in range(_BB)], axis=0)
        drT = jnp.abs(rar_rows - rar3)
        MmT = adjET[None] - adjWT[None] * drT
        obsT = obsT_ref[:, step]
        rz = jnp.concatenate([rar3, zpad], axis=1)
        xhT = jnp.concatenate([obsT, hT, rz], axis=1)
        xhmT = m3 * xhT
        combT = m3 * jnp.stack(
            [jax.lax.dot(xhmT[b], MmT[b], precision=_PREC)
             for b in range(_BB)], axis=0) + xhT
        accT = jnp.stack(
            [jax.lax.dot(
                WruT,
                jnp.concatenate([combT[b]] * _QDIM, axis=0) * qv5,
                precision=_PREC) for b in range(_BB)], axis=0) + bbru[None]
        r = jax.nn.sigmoid(accT[:, :_D])
        u = jax.nn.sigmoid(accT[:, _D:_H2])
        mgt = m3 > 0.0
        h_rT = jnp.where(mgt, r * hT, hT)
        xcT = jnp.concatenate([obsT, h_rT, rz], axis=1)
        candT = jnp.tanh(jnp.stack(
            [jax.lax.dot(
                WcT,
                jnp.concatenate([xcT[b]] * _QDIM, axis=0) * qv5,
                precision=_PREC) for b in range(_BB)], axis=0) + bbc[None])
        h_new = jnp.where(mgt, (1.0 - u) * h_rT + u * candT, hT)
        out_new = jnp.where(lb3 == step + 1, h_new, outT)
        return h_new, out_new

    h0 = jnp.zeros((_BB, _D, _NODES), jnp.float32)
    _, outT = jax.lax.fori_loop(0, _STEPS, step_fn, (h0, h0))
    out_ref[...] = outT


def kernel(obs_emb, observed_mask, lengths, avg_interval, var_plm_rep,
           rarity_W, Wf1, bf1, Wf2, bf2, Wg1, bg1, Wg2, bg2,
           Wu, bu, Wr, br, Wc, bc):
    obsT = obs_emb.transpose(0, 1, 3, 2)
    maskT = (observed_mask.transpose(1, 2, 0)
             .reshape(_STEPS, _NODES, _BATCH // _BB, _BB)
             .transpose(2, 0, 1, 3))
    avgT = (avg_interval.transpose(1, 2, 0)
            .reshape(_STEPS, _NODES, _BATCH // _BB, _BB)
            .transpose(2, 0, 1, 3))
    def _wflat(w):
        wp = jnp.concatenate(
            [w[:, :_D], w[:, _D + 1:], w[:, _D:_D + 1],
             jnp.zeros((_QDIM, _FP - _NF, w.shape[2]), w.dtype)], axis=1)
        return wp.reshape(_QDIM * _FP, w.shape[2]).T

    WruT = _wflat(jnp.stack([Wr, Wu], axis=2).reshape(_QDIM, _NF, 2 * _D))
    WcT = _wflat(Wc)
    bruT = jnp.concatenate([br, bu], axis=1).T
    bcT = bc.T

    full = lambda nd: (lambda i: (0,) * nd)
    outT = pl.pallas_call(
        _rnn_body,
        grid=(_BATCH // _BB,),
        in_specs=[
            pl.BlockSpec((_BB, _STEPS, _D, _NODES), lambda i: (i, 0, 0, 0)),
            pl.BlockSpec((_BB, _STEPS, _NODES), lambda i: (i, 0, 0)),
            pl.BlockSpec((1, _STEPS, _NODES, _BB), lambda i: (i, 0, 0, 0)),
            pl.BlockSpec((_BB, _STEPS, _NODES), lambda i: (i, 0, 0)),
            pl.BlockSpec((1, _STEPS, _NODES, _BB), lambda i: (i, 0, 0, 0)),
            pl.BlockSpec((_BB, 1), lambda i: (i, 0)),
            pl.BlockSpec((_PLM, _NODES), full(2)),
            pl.BlockSpec((_NODES, _NODES), full(2)),
            pl.BlockSpec((_H2, _PLM), full(2)),
            pl.BlockSpec((_H2, 1), full(2)),
            pl.BlockSpec((_QDIM, _H2), full(2)),
            pl.BlockSpec((_QDIM, 1), full(2)),
            pl.BlockSpec((_H2, _PLM), full(2)),
            pl.BlockSpec((_H2, 1), full(2)),
            pl.BlockSpec((8, _H2), full(2)),
            pl.BlockSpec((8, 1), full(2)),
            pl.BlockSpec((2 * _D, _QDIM * _FP), full(2)),
            pl.BlockSpec((_D, _QDIM * _FP), full(2)),
            pl.BlockSpec((2 * _D, _QDIM), full(2)),
            pl.BlockSpec((_D, _QDIM), full(2)),
        ],
        out_specs=pl.BlockSpec((_BB, _D, _NODES), lambda i: (i, 0, 0)),
        out_shape=jax.ShapeDtypeStruct((_BATCH, _D, _NODES), jnp.float32),
        scratch_shapes=[
            pltpu.VMEM((_NODES, _NODES), jnp.float32),
            pltpu.VMEM((_NODES, _NODES), jnp.float32),
            pltpu.VMEM((_QDIM * _FP, _NODES), jnp.float32),
            pltpu.VMEM((2 * _D, _NODES), jnp.float32),
            pltpu.VMEM((_D, _NODES), jnp.float32),
            pltpu.VMEM((_STEPS, _NODES, _BB), jnp.float32),
        ],
        compiler_params=pltpu.CompilerParams(
            dimension_semantics=("arbitrary",)),
    )(obsT, observed_mask, maskT, avg_interval, avgT, lengths,
      var_plm_rep.T, rarity_W.T, Wf1.T, bf1.reshape(-1, 1),
      Wf2.T, bf2.reshape(-1, 1), Wg1.T, bg1.reshape(-1, 1),
      Wg2.T, bg2.reshape(-1, 1), WruT, WcT, bruT, bcT)
    return outT.transpose(0, 2, 1)


# speedup vs baseline: 1.3048x; 1.0264x over previous
import jax
import jax.numpy as jnp
from jax.experimental import pallas as pl
from jax.experimental.pallas import tpu as pltpu

_BATCH, _STEPS, _NODES = 64, 24, 64
_D, _QDIM, _PLM = 32, 5, 768
_ALPHA = 0.5
_BB = 8
_NF = 2 * _D + 1
_FP = 72
_H2 = 2 * _D
_PREC = jax.lax.Precision.DEFAULT


def _rnn_body(obsT_ref, mask_ref, maskT_ref, avgsm_ref, avgT_ref, len_ref,
              vprT_ref, rWT_ref, Wf1T_ref, bf1_ref, Wf2T_ref, bf2_ref,
              Wg1T_ref, bg1_ref, Wg2T_ref, bg2_ref,
              WruT_ref, WcT_ref, bruT_ref, bcT_ref,
              out_ref,
              adjET_s, adjWT_s, qv5_s, bbru_s, bbc_s, rrow_s,
              Mm_s, rl_s):

    @pl.when(pl.program_id(0) == 0)
    def _prologue():
        vprT = vprT_ref[...]
        qhT = jnp.maximum(
            jax.lax.dot(Wf1T_ref[...], vprT, precision=_PREC) + bf1_ref[...],
            0.0)
        qvT = jax.lax.dot(Wf2T_ref[...], qhT, precision=_PREC) + bf2_ref[...]
        ghT = jnp.maximum(
            jax.lax.dot(Wg1T_ref[...], vprT, precision=_PREC) + bg1_ref[...],
            0.0)
        neT = jax.lax.dot(Wg2T_ref[...], ghT, precision=_PREC) + bg2_ref[...]
        nrm = jnp.sqrt(jnp.sum(neT * neT, axis=0, keepdims=True))
        neT = neT / jnp.maximum(nrm, 1e-12)
        logits = jax.lax.dot_general(neT, neT, (((0,), (0,)), ((), ())),
                                     precision=_PREC)
        mx = jnp.max(logits, axis=0, keepdims=True)
        e = jnp.exp(logits - mx)
        adjT = e / jnp.sum(e, axis=0, keepdims=True)
        eye = (jax.lax.broadcasted_iota(jnp.int32, (_NODES, _NODES), 0) ==
               jax.lax.broadcasted_iota(jnp.int32, (_NODES, _NODES), 1)
               ).astype(jnp.float32)
        adjET = adjT * (1.0 - eye)
        adjET_s[...] = adjET
        adjWT_s[...] = adjET * rWT_ref[...]
        qv5_s[...] = jnp.concatenate(
            [jnp.broadcast_to(qvT[d:d + 1, :], (_FP, _NODES))
             for d in range(_QDIM)], axis=0)
        bbru_s[...] = jax.lax.dot(bruT_ref[...], qvT, precision=_PREC)
        bbc_s[...] = jax.lax.dot(bcT_ref[...], qvT, precision=_PREC)

    vto = jnp.sum(mask_ref[...], axis=1)
    vtoT = jnp.sum(maskT_ref[0], axis=0)
    rrow_s[...] = _ALPHA * jnp.tanh(avgT_ref[0] / (vtoT[None] + 1.0))
    lb3 = len_ref[...].reshape(_BB, 1, 1)
    zpad = jnp.zeros((_BB, _FP - _NF, _NODES), jnp.float32)
    adjET = adjET_s[...]
    adjWT = adjWT_s[...]
    qv5 = qv5_s[...]
    bbru = bbru_s[...]
    bbc = bbc_s[...]
    WruT = WruT_ref[...]
    WcT = WcT_ref[...]

    rl_s[...] = _ALPHA * jnp.tanh(avgsm_ref[...] / (vto[None] + 1.0))
    rlane_all = rl_s[...]
    rrow = rrow_s[...]
    rows_all = jnp.stack(
        [rrow[:, :, b:b + 1] for b in range(_BB)], axis=1)
    dr_all = jnp.abs(rows_all - rlane_all[:, :, None, :])
    Mm_s[...] = (adjET[None, None] - adjWT[None, None] * dr_all)

    def step_fn(step, carry):
        hT, outT = carry
        m3 = mask_ref[:, step, :][:, None, :]
        rar3 = rl_s[step][:, None, :]
        MmT = Mm_s[step]
        obsT = obsT_ref[:, step]
        rz = jnp.concatenate([rar3, zpad], axis=1)
        xhT = jnp.concatenate([obsT, hT, rz], axis=1)
        xhmT = m3 * xhT
        combT = m3 * jnp.stack(
            [jax.lax.dot(xhmT[b], MmT[b], precision=_PREC)
             for b in range(_BB)], axis=0) + xhT
        accT = jnp.stack(
            [jax.lax.dot(
                WruT,
                jnp.concatenate([combT[b]] * _QDIM, axis=0) * qv5,
                precision=_PREC) for b in range(_BB)], axis=0) + bbru[None]
        r = jax.nn.sigmoid(accT[:, :_D])
        u = jax.nn.sigmoid(accT[:, _D:_H2])
        mgt = m3 > 0.0
        h_rT = jnp.where(mgt, r * hT, hT)
        xcT = jnp.concatenate([obsT, h_rT, rz], axis=1)
        candT = jnp.tanh(jnp.stack(
            [jax.lax.dot(
                WcT,
                jnp.concatenate([xcT[b]] * _QDIM, axis=0) * qv5,
                precision=_PREC) for b in range(_BB)], axis=0) + bbc[None])
        h_new = jnp.where(mgt, (1.0 - u) * h_rT + u * candT, hT)
        out_new = jnp.where(lb3 == step + 1, h_new, outT)
        return h_new, out_new

    h0 = jnp.zeros((_BB, _D, _NODES), jnp.float32)
    _, outT = jax.lax.fori_loop(0, _STEPS, step_fn, (h0, h0))
    out_ref[...] = outT


def kernel(obs_emb, observed_mask, lengths, avg_interval, var_plm_rep,
           rarity_W, Wf1, bf1, Wf2, bf2, Wg1, bg1, Wg2, bg2,
           Wu, bu, Wr, br, Wc, bc):
    obsT = obs_emb.transpose(0, 1, 3, 2)
    avg_sm = avg_interval.transpose(1, 0, 2)
    maskT = (observed_mask.transpose(1, 2, 0)
             .reshape(_STEPS, _NODES, _BATCH // _BB, _BB)
             .transpose(2, 0, 1, 3))
    avgT = (avg_interval.transpose(1, 2, 0)
            .reshape(_STEPS, _NODES, _BATCH // _BB, _BB)
            .transpose(2, 0, 1, 3))
    def _wflat(w):
        wp = jnp.concatenate(
            [w[:, :_D], w[:, _D + 1:], w[:, _D:_D + 1],
             jnp.zeros((_QDIM, _FP - _NF, w.shape[2]), w.dtype)], axis=1)
        return wp.reshape(_QDIM * _FP, w.shape[2]).T

    WruT = _wflat(jnp.stack([Wr, Wu], axis=2).reshape(_QDIM, _NF, 2 * _D))
    WcT = _wflat(Wc)
    bruT = jnp.concatenate([br, bu], axis=1).T
    bcT = bc.T

    full = lambda nd: (lambda i: (0,) * nd)
    outT = pl.pallas_call(
        _rnn_body,
        grid=(_BATCH // _BB,),
        in_specs=[
            pl.BlockSpec((_BB, _STEPS, _D, _NODES), lambda i: (i, 0, 0, 0)),
            pl.BlockSpec((_BB, _STEPS, _NODES), lambda i: (i, 0, 0)),
            pl.BlockSpec((1, _STEPS, _NODES, _BB), lambda i: (i, 0, 0, 0)),
            pl.BlockSpec((_STEPS, _BB, _NODES), lambda i: (0, i, 0)),
            pl.BlockSpec((1, _STEPS, _NODES, _BB), lambda i: (i, 0, 0, 0)),
            pl.BlockSpec((_BB, 1), lambda i: (i, 0)),
            pl.BlockSpec((_PLM, _NODES), full(2)),
            pl.BlockSpec((_NODES, _NODES), full(2)),
            pl.BlockSpec((_H2, _PLM), full(2)),
            pl.BlockSpec((_H2, 1), full(2)),
            pl.BlockSpec((_QDIM, _H2), full(2)),
            pl.BlockSpec((_QDIM, 1), full(2)),
            pl.BlockSpec((_H2, _PLM), full(2)),
            pl.BlockSpec((_H2, 1), full(2)),
            pl.BlockSpec((8, _H2), full(2)),
            pl.BlockSpec((8, 1), full(2)),
            pl.BlockSpec((2 * _D, _QDIM * _FP), full(2)),
            pl.BlockSpec((_D, _QDIM * _FP), full(2)),
            pl.BlockSpec((2 * _D, _QDIM), full(2)),
            pl.BlockSpec((_D, _QDIM), full(2)),
        ],
        out_specs=pl.BlockSpec((_BB, _D, _NODES), lambda i: (i, 0, 0)),
        out_shape=jax.ShapeDtypeStruct((_BATCH, _D, _NODES), jnp.float32),
        scratch_shapes=[
            pltpu.VMEM((_NODES, _NODES), jnp.float32),
            pltpu.VMEM((_NODES, _NODES), jnp.float32),
            pltpu.VMEM((_QDIM * _FP, _NODES), jnp.float32),
            pltpu.VMEM((2 * _D, _NODES), jnp.float32),
            pltpu.VMEM((_D, _NODES), jnp.float32),
            pltpu.VMEM((_STEPS, _NODES, _BB), jnp.float32),
            pltpu.VMEM((_STEPS, _BB, _NODES, _NODES), jnp.float32),
            pltpu.VMEM((_STEPS, _BB, _NODES), jnp.float32),
        ],
        compiler_params=pltpu.CompilerParams(
            dimension_semantics=("arbitrary",)),
    )(obsT, observed_mask, maskT, avg_sm, avgT, lengths,
      var_plm_rep.T, rarity_W.T, Wf1.T, bf1.reshape(-1, 1),
      Wf2.T, bf2.reshape(-1, 1), Wg1.T, bg1.reshape(-1, 1),
      Wg2.T, bg2.reshape(-1, 1), WruT, WcT, bruT, bcT)
    return outT.transpose(0, 2, 1)


# speedup vs baseline: 1.3110x; 1.0048x over previous
import jax
import jax.numpy as jnp
from jax.experimental import pallas as pl
from jax.experimental.pallas import tpu as pltpu

_BATCH, _STEPS, _NODES = 64, 24, 64
_D, _QDIM, _PLM = 32, 5, 768
_ALPHA = 0.5
_BB = 8
_NF = 2 * _D + 1
_FP = 72
_H2 = 2 * _D
_PREC = jax.lax.Precision.DEFAULT


def _rnn_body(obsT_ref, mask_ref, maskT_ref, avgsm_ref, avgT_ref, len_ref,
              vprT_ref, rWT_ref, Wf1T_ref, bf1_ref, Wf2T_ref, bf2_ref,
              Wg1T_ref, bg1_ref, Wg2T_ref, bg2_ref,
              WruT_ref, WcT_ref, bruT_ref, bcT_ref,
              out_ref,
              adjET_s, adjWT_s, qv5_s, bbru_s, bbc_s, rrow_s,
              Mm_s, rl_s):

    @pl.when(pl.program_id(0) == 0)
    def _prologue():
        vprT = vprT_ref[...]
        qhT = jnp.maximum(
            jax.lax.dot(Wf1T_ref[...], vprT, precision=_PREC) + bf1_ref[...],
            0.0)
        qvT = jax.lax.dot(Wf2T_ref[...], qhT, precision=_PREC) + bf2_ref[...]
        ghT = jnp.maximum(
            jax.lax.dot(Wg1T_ref[...], vprT, precision=_PREC) + bg1_ref[...],
            0.0)
        neT = jax.lax.dot(Wg2T_ref[...], ghT, precision=_PREC) + bg2_ref[...]
        nrm = jnp.sqrt(jnp.sum(neT * neT, axis=0, keepdims=True))
        neT = neT / jnp.maximum(nrm, 1e-12)
        logits = jax.lax.dot_general(neT, neT, (((0,), (0,)), ((), ())),
                                     precision=_PREC)
        mx = jnp.max(logits, axis=0, keepdims=True)
        e = jnp.exp(logits - mx)
        adjT = e / jnp.sum(e, axis=0, keepdims=True)
        eye = (jax.lax.broadcasted_iota(jnp.int32, (_NODES, _NODES), 0) ==
               jax.lax.broadcasted_iota(jnp.int32, (_NODES, _NODES), 1)
               ).astype(jnp.float32)
        adjET = adjT * (1.0 - eye)
        adjET_s[...] = adjET
        adjWT_s[...] = adjET * rWT_ref[...]
        qv5_s[...] = jnp.concatenate(
            [jnp.broadcast_to(qvT[d:d + 1, :], (_FP, _NODES))
             for d in range(_QDIM)], axis=0)
        bbru_s[...] = jax.lax.dot(bruT_ref[...], qvT, precision=_PREC)
        bbc_s[...] = jax.lax.dot(bcT_ref[...], qvT, precision=_PREC)

    vto = jnp.sum(mask_ref[...], axis=1)
    vtoT = jnp.sum(maskT_ref[0], axis=0)
    rrow_s[...] = _ALPHA * jnp.tanh(avgT_ref[0] / (vtoT[None] + 1.0))
    lb3 = len_ref[...].reshape(_BB, 1, 1)
    zpad = jnp.zeros((_BB, _FP - _NF, _NODES), jnp.float32)
    adjET = adjET_s[...]
    adjWT = adjWT_s[...]
    qv5 = qv5_s[...]
    bbru = bbru_s[...]
    bbc = bbc_s[...]
    WruT = WruT_ref[...]
    WcT = WcT_ref[...]

    rl_s[...] = _ALPHA * jnp.tanh(avgsm_ref[...] / (vto[None] + 1.0))
    rlane_all = rl_s[...]
    rrow = rrow_s[...]
    rows_all = jnp.stack(
        [rrow[:, :, b:b + 1] for b in range(_BB)], axis=1)
    dr_all = jnp.abs(rows_all - rlane_all[:, :, None, :])
    Mm_s[...] = (adjET[None, None] - adjWT[None, None] * dr_all
                 ).astype(jnp.bfloat16)

    def step_fn(step, carry):
        hT, outT = carry
        m3 = mask_ref[:, step, :][:, None, :]
        rar3 = rl_s[step][:, None, :]
        MmT = Mm_s[step]
        obsT = obsT_ref[:, step]
        rz = jnp.concatenate([rar3, zpad], axis=1)
        xhT = jnp.concatenate([obsT, hT, rz], axis=1)
        xhmT = (m3 * xhT).astype(jnp.bfloat16)
        combT = m3 * jnp.stack(
            [jax.lax.dot(xhmT[b], MmT[b], precision=_PREC,
                         preferred_element_type=jnp.float32)
             for b in range(_BB)], axis=0) + xhT
        accT = jnp.stack(
            [jax.lax.dot(
                WruT,
                (jnp.concatenate([combT[b]] * _QDIM, axis=0)
                 * qv5).astype(jnp.bfloat16),
                precision=_PREC,
                preferred_element_type=jnp.float32)
             for b in range(_BB)], axis=0) + bbru[None]
        r = jax.nn.sigmoid(accT[:, :_D])
        u = jax.nn.sigmoid(accT[:, _D:_H2])
        mgt = m3 > 0.0
        h_rT = jnp.where(mgt, r * hT, hT)
        xcT = jnp.concatenate([obsT, h_rT, rz], axis=1)
        candT = jnp.tanh(jnp.stack(
            [jax.lax.dot(
                WcT,
                (jnp.concatenate([xcT[b]] * _QDIM, axis=0)
                 * qv5).astype(jnp.bfloat16),
                precision=_PREC,
                preferred_element_type=jnp.float32)
             for b in range(_BB)], axis=0) + bbc[None])
        h_new = jnp.where(mgt, (1.0 - u) * h_rT + u * candT, hT)
        out_new = jnp.where(lb3 == step + 1, h_new, outT)
        return h_new, out_new

    h0 = jnp.zeros((_BB, _D, _NODES), jnp.float32)
    _, outT = jax.lax.fori_loop(0, _STEPS, step_fn, (h0, h0))
    out_ref[...] = outT


def kernel(obs_emb, observed_mask, lengths, avg_interval, var_plm_rep,
           rarity_W, Wf1, bf1, Wf2, bf2, Wg1, bg1, Wg2, bg2,
           Wu, bu, Wr, br, Wc, bc):
    obsT = obs_emb.transpose(0, 1, 3, 2)
    avg_sm = avg_interval.transpose(1, 0, 2)
    maskT = (observed_mask.transpose(1, 2, 0)
             .reshape(_STEPS, _NODES, _BATCH // _BB, _BB)
             .transpose(2, 0, 1, 3))
    avgT = (avg_interval.transpose(1, 2, 0)
            .reshape(_STEPS, _NODES, _BATCH // _BB, _BB)
            .transpose(2, 0, 1, 3))
    def _wflat(w):
        wp = jnp.concatenate(
            [w[:, :_D], w[:, _D + 1:], w[:, _D:_D + 1],
             jnp.zeros((_QDIM, _FP - _NF, w.shape[2]), w.dtype)], axis=1)
        return wp.reshape(_QDIM * _FP, w.shape[2]).T

    WruT = _wflat(jnp.stack([Wr, Wu], axis=2).reshape(_QDIM, _NF, 2 * _D))
    WcT = _wflat(Wc)
    bruT = jnp.concatenate([br, bu], axis=1).T
    bcT = bc.T

    full = lambda nd: (lambda i: (0,) * nd)
    outT = pl.pallas_call(
        _rnn_body,
        grid=(_BATCH // _BB,),
        in_specs=[
            pl.BlockSpec((_BB, _STEPS, _D, _NODES), lambda i: (i, 0, 0, 0)),
            pl.BlockSpec((_BB, _STEPS, _NODES), lambda i: (i, 0, 0)),
            pl.BlockSpec((1, _STEPS, _NODES, _BB), lambda i: (i, 0, 0, 0)),
            pl.BlockSpec((_STEPS, _BB, _NODES), lambda i: (0, i, 0)),
            pl.BlockSpec((1, _STEPS, _NODES, _BB), lambda i: (i, 0, 0, 0)),
            pl.BlockSpec((_BB, 1), lambda i: (i, 0)),
            pl.BlockSpec((_PLM, _NODES), full(2)),
            pl.BlockSpec((_NODES, _NODES), full(2)),
            pl.BlockSpec((_H2, _PLM), full(2)),
            pl.BlockSpec((_H2, 1), full(2)),
            pl.BlockSpec((_QDIM, _H2), full(2)),
            pl.BlockSpec((_QDIM, 1), full(2)),
            pl.BlockSpec((_H2, _PLM), full(2)),
            pl.BlockSpec((_H2, 1), full(2)),
            pl.BlockSpec((8, _H2), full(2)),
            pl.BlockSpec((8, 1), full(2)),
            pl.BlockSpec((2 * _D, _QDIM * _FP), full(2)),
            pl.BlockSpec((_D, _QDIM * _FP), full(2)),
            pl.BlockSpec((2 * _D, _QDIM), full(2)),
            pl.BlockSpec((_D, _QDIM), full(2)),
        ],
        out_specs=pl.BlockSpec((_BB, _D, _NODES), lambda i: (i, 0, 0)),
        out_shape=jax.ShapeDtypeStruct((_BATCH, _D, _NODES), jnp.float32),
        scratch_shapes=[
            pltpu.VMEM((_NODES, _NODES), jnp.float32),
            pltpu.VMEM((_NODES, _NODES), jnp.float32),
            pltpu.VMEM((_QDIM * _FP, _NODES), jnp.float32),
            pltpu.VMEM((2 * _D, _NODES), jnp.float32),
            pltpu.VMEM((_D, _NODES), jnp.float32),
            pltpu.VMEM((_STEPS, _NODES, _BB), jnp.float32),
            pltpu.VMEM((_STEPS, _BB, _NODES, _NODES), jnp.bfloat16),
            pltpu.VMEM((_STEPS, _BB, _NODES), jnp.float32),
        ],
        compiler_params=pltpu.CompilerParams(
            dimension_semantics=("arbitrary",)),
    )(obsT, observed_mask, maskT, avg_sm, avgT, lengths,
      var_plm_rep.T, rarity_W.T, Wf1.T, bf1.reshape(-1, 1),
      Wf2.T, bf2.reshape(-1, 1), Wg1.T, bg1.reshape(-1, 1),
      Wg2.T, bg2.reshape(-1, 1), WruT.astype(jnp.bfloat16),
      WcT.astype(jnp.bfloat16), bruT, bcT)
    return outT.transpose(0, 2, 1)


# speedup vs baseline: 1.5545x; 1.1857x over previous
import jax
import jax.numpy as jnp
from jax.experimental import pallas as pl
from jax.experimental.pallas import tpu as pltpu

_BATCH, _STEPS, _NODES = 64, 24, 64
_D, _QDIM, _PLM = 32, 5, 768
_ALPHA = 0.5
_BB = 16
_NF = 2 * _D + 1
_FP = 72
_H2 = 2 * _D
_PREC = jax.lax.Precision.DEFAULT


def _rnn_body(obsT_ref, mask_ref, maskT_ref, avgsm_ref, avgT_ref, len_ref,
              vprT_ref, rWT_ref, Wf1T_ref, bf1_ref, Wf2T_ref, bf2_ref,
              Wg1T_ref, bg1_ref, Wg2T_ref, bg2_ref,
              WruT_ref, WcT_ref, bruT_ref, bcT_ref,
              out_ref,
              adjET_s, adjWT_s, qv5_s, bbru_s, bbc_s, rrow_s,
              Mm_s, rl_s):

    @pl.when(pl.program_id(0) == 0)
    def _prologue():
        vprT = vprT_ref[...]
        qhT = jnp.maximum(
            jax.lax.dot(Wf1T_ref[...], vprT, precision=_PREC) + bf1_ref[...],
            0.0)
        qvT = jax.lax.dot(Wf2T_ref[...], qhT, precision=_PREC) + bf2_ref[...]
        ghT = jnp.maximum(
            jax.lax.dot(Wg1T_ref[...], vprT, precision=_PREC) + bg1_ref[...],
            0.0)
        neT = jax.lax.dot(Wg2T_ref[...], ghT, precision=_PREC) + bg2_ref[...]
        nrm = jnp.sqrt(jnp.sum(neT * neT, axis=0, keepdims=True))
        neT = neT / jnp.maximum(nrm, 1e-12)
        logits = jax.lax.dot_general(neT, neT, (((0,), (0,)), ((), ())),
                                     precision=_PREC)
        mx = jnp.max(logits, axis=0, keepdims=True)
        e = jnp.exp(logits - mx)
        adjT = e / jnp.sum(e, axis=0, keepdims=True)
        eye = (jax.lax.broadcasted_iota(jnp.int32, (_NODES, _NODES), 0) ==
               jax.lax.broadcasted_iota(jnp.int32, (_NODES, _NODES), 1)
               ).astype(jnp.float32)
        adjET = adjT * (1.0 - eye)
        adjET_s[...] = adjET
        adjWT_s[...] = adjET * rWT_ref[...]
        qv5_s[...] = jnp.concatenate(
            [jnp.broadcast_to(qvT[d:d + 1, :], (_FP, _NODES))
             for d in range(_QDIM)], axis=0)
        bbru_s[...] = jax.lax.dot(bruT_ref[...], qvT, precision=_PREC)
        bbc_s[...] = jax.lax.dot(bcT_ref[...], qvT, precision=_PREC)

    vto = jnp.sum(mask_ref[...], axis=1)
    vtoT = jnp.sum(maskT_ref[0], axis=0)
    rrow_s[...] = _ALPHA * jnp.tanh(avgT_ref[0] / (vtoT[None] + 1.0))
    lb3 = len_ref[...].reshape(_BB, 1, 1)
    zpad = jnp.zeros((_BB, _FP - _NF, _NODES), jnp.float32)
    adjET = adjET_s[...]
    adjWT = adjWT_s[...]
    qv5 = qv5_s[...]
    bbru = bbru_s[...]
    bbc = bbc_s[...]
    WruT = WruT_ref[...]
    WcT = WcT_ref[...]

    rl_s[...] = _ALPHA * jnp.tanh(avgsm_ref[...] / (vto[None] + 1.0))
    rlane_all = rl_s[...]
    rrow = rrow_s[...]
    rows_all = jnp.stack(
        [rrow[:, :, b:b + 1] for b in range(_BB)], axis=1)
    dr_all = jnp.abs(rows_all - rlane_all[:, :, None, :])
    Mm_s[...] = (adjET[None, None] - adjWT[None, None] * dr_all
                 ).astype(jnp.bfloat16)

    def step_fn(step, carry):
        hT, outT = carry
        m3 = mask_ref[:, step, :][:, None, :]
        rar3 = rl_s[step][:, None, :]
        MmT = Mm_s[step]
        obsT = obsT_ref[:, step]
        rz = jnp.concatenate([rar3, zpad], axis=1)
        xhT = jnp.concatenate([obsT, hT, rz], axis=1)
        xhmT = (m3 * xhT).astype(jnp.bfloat16)
        combT = m3 * jnp.stack(
            [jax.lax.dot(xhmT[b], MmT[b], precision=_PREC,
                         preferred_element_type=jnp.float32)
             for b in range(_BB)], axis=0) + xhT
        accT = jnp.stack(
            [jax.lax.dot(
                WruT,
                (jnp.concatenate([combT[b]] * _QDIM, axis=0)
                 * qv5).astype(jnp.bfloat16),
                precision=_PREC,
                preferred_element_type=jnp.float32)
             for b in range(_BB)], axis=0) + bbru[None]
        r = jax.nn.sigmoid(accT[:, :_D])
        u = jax.nn.sigmoid(accT[:, _D:_H2])
        mgt = m3 > 0.0
        h_rT = jnp.where(mgt, r * hT, hT)
        xcT = jnp.concatenate([obsT, h_rT, rz], axis=1)
        candT = jnp.tanh(jnp.stack(
            [jax.lax.dot(
                WcT,
                (jnp.concatenate([xcT[b]] * _QDIM, axis=0)
                 * qv5).astype(jnp.bfloat16),
                precision=_PREC,
                preferred_element_type=jnp.float32)
             for b in range(_BB)], axis=0) + bbc[None])
        h_new = jnp.where(mgt, (1.0 - u) * h_rT + u * candT, hT)
        out_new = jnp.where(lb3 == step + 1, h_new, outT)
        return h_new, out_new

    h0 = jnp.zeros((_BB, _D, _NODES), jnp.float32)
    _, outT = jax.lax.fori_loop(0, _STEPS, step_fn, (h0, h0))
    out_ref[...] = outT


def kernel(obs_emb, observed_mask, lengths, avg_interval, var_plm_rep,
           rarity_W, Wf1, bf1, Wf2, bf2, Wg1, bg1, Wg2, bg2,
           Wu, bu, Wr, br, Wc, bc):
    obsT = obs_emb.transpose(0, 1, 3, 2)
    avg_sm = avg_interval.transpose(1, 0, 2)
    maskT = (observed_mask.transpose(1, 2, 0)
             .reshape(_STEPS, _NODES, _BATCH // _BB, _BB)
             .transpose(2, 0, 1, 3))
    avgT = (avg_interval.transpose(1, 2, 0)
            .reshape(_STEPS, _NODES, _BATCH // _BB, _BB)
            .transpose(2, 0, 1, 3))
    def _wflat(w):
        wp = jnp.concatenate(
            [w[:, :_D], w[:, _D + 1:], w[:, _D:_D + 1],
             jnp.zeros((_QDIM, _FP - _NF, w.shape[2]), w.dtype)], axis=1)
        return wp.reshape(_QDIM * _FP, w.shape[2]).T

    WruT = _wflat(jnp.stack([Wr, Wu], axis=2).reshape(_QDIM, _NF, 2 * _D))
    WcT = _wflat(Wc)
    bruT = jnp.concatenate([br, bu], axis=1).T
    bcT = bc.T

    full = lambda nd: (lambda i: (0,) * nd)
    outT = pl.pallas_call(
        _rnn_body,
        grid=(_BATCH // _BB,),
        in_specs=[
            pl.BlockSpec((_BB, _STEPS, _D, _NODES), lambda i: (i, 0, 0, 0)),
            pl.BlockSpec((_BB, _STEPS, _NODES), lambda i: (i, 0, 0)),
            pl.BlockSpec((1, _STEPS, _NODES, _BB), lambda i: (i, 0, 0, 0)),
            pl.BlockSpec((_STEPS, _BB, _NODES), lambda i: (0, i, 0)),
            pl.BlockSpec((1, _STEPS, _NODES, _BB), lambda i: (i, 0, 0, 0)),
            pl.BlockSpec((_BB, 1), lambda i: (i, 0)),
            pl.BlockSpec((_PLM, _NODES), full(2)),
            pl.BlockSpec((_NODES, _NODES), full(2)),
            pl.BlockSpec((_H2, _PLM), full(2)),
            pl.BlockSpec((_H2, 1), full(2)),
            pl.BlockSpec((_QDIM, _H2), full(2)),
            pl.BlockSpec((_QDIM, 1), full(2)),
            pl.BlockSpec((_H2, _PLM), full(2)),
            pl.BlockSpec((_H2, 1), full(2)),
            pl.BlockSpec((8, _H2), full(2)),
            pl.BlockSpec((8, 1), full(2)),
            pl.BlockSpec((2 * _D, _QDIM * _FP), full(2)),
            pl.BlockSpec((_D, _QDIM * _FP), full(2)),
            pl.BlockSpec((2 * _D, _QDIM), full(2)),
            pl.BlockSpec((_D, _QDIM), full(2)),
        ],
        out_specs=pl.BlockSpec((_BB, _D, _NODES), lambda i: (i, 0, 0)),
        out_shape=jax.ShapeDtypeStruct((_BATCH, _D, _NODES), jnp.float32),
        scratch_shapes=[
            pltpu.VMEM((_NODES, _NODES), jnp.float32),
            pltpu.VMEM((_NODES, _NODES), jnp.float32),
            pltpu.VMEM((_QDIM * _FP, _NODES), jnp.float32),
            pltpu.VMEM((2 * _D, _NODES), jnp.float32),
            pltpu.VMEM((_D, _NODES), jnp.float32),
            pltpu.VMEM((_STEPS, _NODES, _BB), jnp.float32),
            pltpu.VMEM((_STEPS, _BB, _NODES, _NODES), jnp.bfloat16),
            pltpu.VMEM((_STEPS, _BB, _NODES), jnp.float32),
        ],
        compiler_params=pltpu.CompilerParams(
            dimension_semantics=("arbitrary",)),
    )(obsT, observed_mask, maskT, avg_sm, avgT, lengths,
      var_plm_rep.T, rarity_W.T, Wf1.T, bf1.reshape(-1, 1),
      Wf2.T, bf2.reshape(-1, 1), Wg1.T, bg1.reshape(-1, 1),
      Wg2.T, bg2.reshape(-1, 1), WruT.astype(jnp.bfloat16),
      WcT.astype(jnp.bfloat16), bruT, bcT)
    return outT.transpose(0, 2, 1)


# speedup vs baseline: 1.6950x; 1.0904x over previous
import jax
import jax.numpy as jnp
from jax.experimental import pallas as pl
from jax.experimental.pallas import tpu as pltpu

_BATCH, _STEPS, _NODES = 64, 24, 64
_D, _QDIM, _PLM = 32, 5, 768
_ALPHA = 0.5
_BB = 16
_NF = 2 * _D + 1
_FP = 72
_H2 = 2 * _D
_PREC = jax.lax.Precision.DEFAULT


def _rnn_body(obsT_ref, mask_ref, maskT_ref, avgsm_ref, avgT_ref, len_ref,
              vprT_ref, rWT_ref, Wf1T_ref, bf1_ref, Wf2T_ref, bf2_ref,
              Wg1T_ref, bg1_ref, Wg2T_ref, bg2_ref,
              WruT_ref, WcT_ref, bruT_ref, bcT_ref,
              out_ref,
              adjET_s, adjWT_s, qv5_s, bbru_s, bbc_s, rrow_s,
              Mm_s, rl_s):

    @pl.when(pl.program_id(0) == 0)
    def _prologue():
        vprT = vprT_ref[...]
        qhT = jnp.maximum(
            jax.lax.dot(Wf1T_ref[...], vprT, precision=_PREC) + bf1_ref[...],
            0.0)
        qvT = jax.lax.dot(Wf2T_ref[...], qhT, precision=_PREC) + bf2_ref[...]
        ghT = jnp.maximum(
            jax.lax.dot(Wg1T_ref[...], vprT, precision=_PREC) + bg1_ref[...],
            0.0)
        neT = jax.lax.dot(Wg2T_ref[...], ghT, precision=_PREC) + bg2_ref[...]
        nrm = jnp.sqrt(jnp.sum(neT * neT, axis=0, keepdims=True))
        neT = neT / jnp.maximum(nrm, 1e-12)
        logits = jax.lax.dot_general(neT, neT, (((0,), (0,)), ((), ())),
                                     precision=_PREC)
        mx = jnp.max(logits, axis=0, keepdims=True)
        e = jnp.exp(logits - mx)
        adjT = e / jnp.sum(e, axis=0, keepdims=True)
        eye = (jax.lax.broadcasted_iota(jnp.int32, (_NODES, _NODES), 0) ==
               jax.lax.broadcasted_iota(jnp.int32, (_NODES, _NODES), 1)
               ).astype(jnp.float32)
        adjET = adjT * (1.0 - eye)
        adjET_s[...] = adjET
        adjWT_s[...] = adjET * rWT_ref[...]
        qv5_s[...] = jnp.concatenate(
            [jnp.broadcast_to(qvT[d:d + 1, :], (_FP, _NODES))
             for d in range(_QDIM)], axis=0).astype(jnp.bfloat16)
        bbru_s[...] = jax.lax.dot(bruT_ref[...], qvT, precision=_PREC)
        bbc_s[...] = jax.lax.dot(bcT_ref[...], qvT, precision=_PREC)

    vto = jnp.sum(mask_ref[...], axis=1)
    vtoT = jnp.sum(maskT_ref[0], axis=0)
    rrow_s[...] = _ALPHA * jnp.tanh(avgT_ref[0] / (vtoT[None] + 1.0))
    lb3 = len_ref[...].reshape(_BB, 1, 1)
    zpad = jnp.zeros((_BB, _FP - _NF, _NODES), jnp.float32)
    adjET = adjET_s[...]
    adjWT = adjWT_s[...]
    qv5 = qv5_s[...]
    bbru = bbru_s[...]
    bbc = bbc_s[...]
    WruT = WruT_ref[...]
    WcT = WcT_ref[...]

    rl_s[...] = _ALPHA * jnp.tanh(avgsm_ref[...] / (vto[None] + 1.0))
    rlane_all = rl_s[...]
    rrow = rrow_s[...]
    rows_all = jnp.stack(
        [rrow[:, :, b:b + 1] for b in range(_BB)], axis=1)
    dr_all = jnp.abs(rows_all - rlane_all[:, :, None, :])
    Mm_s[...] = (adjET[None, None] - adjWT[None, None] * dr_all
                 ).astype(jnp.bfloat16)

    def step_fn(step, carry):
        hT, outT = carry
        m3 = mask_ref[:, step, :][:, None, :]
        rar3 = rl_s[step][:, None, :]
        MmT = Mm_s[step]
        obsT = obsT_ref[:, step]
        rz = jnp.concatenate([rar3, zpad], axis=1)
        xhT = jnp.concatenate([obsT, hT, rz], axis=1)
        xhmT = (m3 * xhT).astype(jnp.bfloat16)
        combT = (m3 * jnp.stack(
            [jax.lax.dot(xhmT[b], MmT[b], precision=_PREC,
                         preferred_element_type=jnp.float32)
             for b in range(_BB)], axis=0) + xhT).astype(jnp.bfloat16)
        accT = jnp.stack(
            [jax.lax.dot(
                WruT,
                jnp.concatenate([combT[b]] * _QDIM, axis=0) * qv5,
                precision=_PREC,
                preferred_element_type=jnp.float32)
             for b in range(_BB)], axis=0) + bbru[None]
        r = jax.nn.sigmoid(accT[:, :_D])
        u = jax.nn.sigmoid(accT[:, _D:_H2])
        mgt = m3 > 0.0
        h_rT = jnp.where(mgt, r * hT, hT)
        xcT = jnp.concatenate(
            [obsT, h_rT, rz], axis=1).astype(jnp.bfloat16)
        candT = jnp.tanh(jnp.stack(
            [jax.lax.dot(
                WcT,
                jnp.concatenate([xcT[b]] * _QDIM, axis=0) * qv5,
                precision=_PREC,
                preferred_element_type=jnp.float32)
             for b in range(_BB)], axis=0) + bbc[None])
        h_new = jnp.where(mgt, (1.0 - u) * h_rT + u * candT, hT)
        out_new = jnp.where(lb3 == step + 1, h_new, outT)
        return h_new, out_new

    h0 = jnp.zeros((_BB, _D, _NODES), jnp.float32)
    _, outT = jax.lax.fori_loop(0, _STEPS, step_fn, (h0, h0))
    out_ref[...] = outT


def kernel(obs_emb, observed_mask, lengths, avg_interval, var_plm_rep,
           rarity_W, Wf1, bf1, Wf2, bf2, Wg1, bg1, Wg2, bg2,
           Wu, bu, Wr, br, Wc, bc):
    obsT = obs_emb.transpose(0, 1, 3, 2)
    avg_sm = avg_interval.transpose(1, 0, 2)
    maskT = (observed_mask.transpose(1, 2, 0)
             .reshape(_STEPS, _NODES, _BATCH // _BB, _BB)
             .transpose(2, 0, 1, 3))
    avgT = (avg_interval.transpose(1, 2, 0)
            .reshape(_STEPS, _NODES, _BATCH // _BB, _BB)
            .transpose(2, 0, 1, 3))
    def _wflat(w):
        wp = jnp.concatenate(
            [w[:, :_D], w[:, _D + 1:], w[:, _D:_D + 1],
             jnp.zeros((_QDIM, _FP - _NF, w.shape[2]), w.dtype)], axis=1)
        return wp.reshape(_QDIM * _FP, w.shape[2]).T

    WruT = _wflat(jnp.stack([Wr, Wu], axis=2).reshape(_QDIM, _NF, 2 * _D))
    WcT = _wflat(Wc)
    bruT = jnp.concatenate([br, bu], axis=1).T
    bcT = bc.T

    full = lambda nd: (lambda i: (0,) * nd)
    outT = pl.pallas_call(
        _rnn_body,
        grid=(_BATCH // _BB,),
        in_specs=[
            pl.BlockSpec((_BB, _STEPS, _D, _NODES), lambda i: (i, 0, 0, 0)),
            pl.BlockSpec((_BB, _STEPS, _NODES), lambda i: (i, 0, 0)),
            pl.BlockSpec((1, _STEPS, _NODES, _BB), lambda i: (i, 0, 0, 0)),
            pl.BlockSpec((_STEPS, _BB, _NODES), lambda i: (0, i, 0)),
            pl.BlockSpec((1, _STEPS, _NODES, _BB), lambda i: (i, 0, 0, 0)),
            pl.BlockSpec((_BB, 1), lambda i: (i, 0)),
            pl.BlockSpec((_PLM, _NODES), full(2)),
            pl.BlockSpec((_NODES, _NODES), full(2)),
            pl.BlockSpec((_H2, _PLM), full(2)),
            pl.BlockSpec((_H2, 1), full(2)),
            pl.BlockSpec((_QDIM, _H2), full(2)),
            pl.BlockSpec((_QDIM, 1), full(2)),
            pl.BlockSpec((_H2, _PLM), full(2)),
            pl.BlockSpec((_H2, 1), full(2)),
            pl.BlockSpec((8, _H2), full(2)),
            pl.BlockSpec((8, 1), full(2)),
            pl.BlockSpec((2 * _D, _QDIM * _FP), full(2)),
            pl.BlockSpec((_D, _QDIM * _FP), full(2)),
            pl.BlockSpec((2 * _D, _QDIM), full(2)),
            pl.BlockSpec((_D, _QDIM), full(2)),
        ],
        out_specs=pl.BlockSpec((_BB, _D, _NODES), lambda i: (i, 0, 0)),
        out_shape=jax.ShapeDtypeStruct((_BATCH, _D, _NODES), jnp.float32),
        scratch_shapes=[
            pltpu.VMEM((_NODES, _NODES), jnp.float32),
            pltpu.VMEM((_NODES, _NODES), jnp.float32),
            pltpu.VMEM((_QDIM * _FP, _NODES), jnp.bfloat16),
            pltpu.VMEM((2 * _D, _NODES), jnp.float32),
            pltpu.VMEM((_D, _NODES), jnp.float32),
            pltpu.VMEM((_STEPS, _NODES, _BB), jnp.float32),
            pltpu.VMEM((_STEPS, _BB, _NODES, _NODES), jnp.bfloat16),
            pltpu.VMEM((_STEPS, _BB, _NODES), jnp.float32),
        ],
        compiler_params=pltpu.CompilerParams(
            dimension_semantics=("arbitrary",)),
    )(obsT, observed_mask, maskT, avg_sm, avgT, lengths,
      var_plm_rep.T, rarity_W.T, Wf1.T, bf1.reshape(-1, 1),
      Wf2.T, bf2.reshape(-1, 1), Wg1.T, bg1.reshape(-1, 1),
      Wg2.T, bg2.reshape(-1, 1), WruT.astype(jnp.bfloat16),
      WcT.astype(jnp.bfloat16), bruT, bcT)
    return outT.transpose(0, 2, 1)


# speedup vs baseline: 1.8613x; 1.0981x over previous
import jax
import jax.numpy as jnp
from jax.experimental import pallas as pl
from jax.experimental.pallas import tpu as pltpu

_BATCH, _STEPS, _NODES = 64, 24, 64
_D, _QDIM, _PLM = 32, 5, 768
_ALPHA = 0.5
_BB = 16
_NF = 2 * _D + 1
_FP = 72
_H2 = 2 * _D
_PREC = jax.lax.Precision.DEFAULT


def _rnn_body(obsT_ref, mask_ref, maskT_ref, avgsm_ref, avgT_ref, avgb_ref,
              len_ref,
              vprT_ref, rWT_ref, Wf1T_ref, bf1_ref, Wf2T_ref, bf2_ref,
              Wg1T_ref, bg1_ref, Wg2T_ref, bg2_ref,
              WruT_ref, WcT_ref, bruT_ref, bcT_ref,
              out_ref,
              adjET_s, adjWT_s, qv5_s, bbru_s, bbc_s, rrow_s,
              Mm_s, rl_s):

    @pl.when(pl.program_id(0) == 0)
    def _prologue():
        vprT = vprT_ref[...]
        qhT = jnp.maximum(
            jax.lax.dot(Wf1T_ref[...], vprT, precision=_PREC) + bf1_ref[...],
            0.0)
        qvT = jax.lax.dot(Wf2T_ref[...], qhT, precision=_PREC) + bf2_ref[...]
        ghT = jnp.maximum(
            jax.lax.dot(Wg1T_ref[...], vprT, precision=_PREC) + bg1_ref[...],
            0.0)
        neT = jax.lax.dot(Wg2T_ref[...], ghT, precision=_PREC) + bg2_ref[...]
        nrm = jnp.sqrt(jnp.sum(neT * neT, axis=0, keepdims=True))
        neT = neT / jnp.maximum(nrm, 1e-12)
        logits = jax.lax.dot_general(neT, neT, (((0,), (0,)), ((), ())),
                                     precision=_PREC)
        mx = jnp.max(logits, axis=0, keepdims=True)
        e = jnp.exp(logits - mx)
        adjT = e / jnp.sum(e, axis=0, keepdims=True)
        eye = (jax.lax.broadcasted_iota(jnp.int32, (_NODES, _NODES), 0) ==
               jax.lax.broadcasted_iota(jnp.int32, (_NODES, _NODES), 1)
               ).astype(jnp.float32)
        adjET = adjT * (1.0 - eye)
        adjET_s[...] = adjET
        adjWT_s[...] = adjET * rWT_ref[...]
        qv5_s[...] = jnp.concatenate(
            [jnp.broadcast_to(qvT[d:d + 1, :], (_FP, _NODES))
             for d in range(_QDIM)], axis=0).astype(jnp.bfloat16)
        bbru_s[...] = jax.lax.dot(bruT_ref[...], qvT, precision=_PREC)
        bbc_s[...] = jax.lax.dot(bcT_ref[...], qvT, precision=_PREC)

    vto = jnp.sum(mask_ref[...], axis=1)
    vtoT = jnp.sum(maskT_ref[0], axis=0)
    rrow_s[...] = _ALPHA * jnp.tanh(avgT_ref[0] / (vtoT[None] + 1.0))
    lb3 = len_ref[...].reshape(_BB, 1, 1)
    zpad = jnp.zeros((_BB, _FP - _NF, _NODES), jnp.float32)
    adjET = adjET_s[...]
    adjWT = adjWT_s[...]
    qv5 = qv5_s[...]
    bbru = bbru_s[...]
    bbc = bbc_s[...]
    WruT = WruT_ref[...]
    WcT = WcT_ref[...]

    rl_s[...] = _ALPHA * jnp.tanh(avgsm_ref[...] / (vto[None] + 1.0))
    rlb = _ALPHA * jnp.tanh(avgb_ref[...] / (vto[:, None, :] + 1.0))
    r2d = rrow_s[...].reshape(_STEPS * _NODES, _BB)
    adjE_t = jnp.concatenate([adjET] * _STEPS, axis=0)
    adjW_t = jnp.concatenate([adjWT] * _STEPS, axis=0)
    bio = jax.lax.broadcasted_iota(jnp.int32, (_BB, _NODES), 0)
    for b in range(_BB):
        sel = (bio == b).astype(jnp.float32)
        rows_b = jax.lax.dot(r2d, sel, precision=_PREC)
        cols_b = jnp.broadcast_to(
            rlb[b][:, None, :], (_STEPS, _NODES, _NODES)
        ).reshape(_STEPS * _NODES, _NODES)
        Mm_s[:, b] = (adjE_t - adjW_t * jnp.abs(rows_b - cols_b)
                      ).reshape(_STEPS, _NODES, _NODES).astype(jnp.bfloat16)

    def step_fn(step, carry):
        hT, outT = carry
        m3 = mask_ref[:, step, :][:, None, :]
        rar3 = rl_s[step][:, None, :]
        MmT = Mm_s[step]
        obsT = obsT_ref[:, step]
        rz = jnp.concatenate([rar3, zpad], axis=1)
        xhT = jnp.concatenate([obsT, hT, rz], axis=1)
        xhmT = (m3 * xhT).astype(jnp.bfloat16)
        combT = (m3 * jnp.stack(
            [jax.lax.dot(xhmT[b], MmT[b], precision=_PREC,
                         preferred_element_type=jnp.float32)
             for b in range(_BB)], axis=0) + xhT).astype(jnp.bfloat16)
        accT = jnp.stack(
            [jax.lax.dot(
                WruT,
                jnp.concatenate([combT[b]] * _QDIM, axis=0) * qv5,
                precision=_PREC,
                preferred_element_type=jnp.float32)
             for b in range(_BB)], axis=0) + bbru[None]
        r = jax.nn.sigmoid(accT[:, :_D])
        u = jax.nn.sigmoid(accT[:, _D:_H2])
        mgt = m3 > 0.0
        h_rT = jnp.where(mgt, r * hT, hT)
        xcT = jnp.concatenate(
            [obsT, h_rT, rz], axis=1).astype(jnp.bfloat16)
        candT = jnp.tanh(jnp.stack(
            [jax.lax.dot(
                WcT,
                jnp.concatenate([xcT[b]] * _QDIM, axis=0) * qv5,
                precision=_PREC,
                preferred_element_type=jnp.float32)
             for b in range(_BB)], axis=0) + bbc[None])
        h_new = jnp.where(mgt, (1.0 - u) * h_rT + u * candT, hT)
        out_new = jnp.where(lb3 == step + 1, h_new, outT)
        return h_new, out_new

    h0 = jnp.zeros((_BB, _D, _NODES), jnp.float32)
    _, outT = jax.lax.fori_loop(0, _STEPS, step_fn, (h0, h0))
    out_ref[...] = outT


def kernel(obs_emb, observed_mask, lengths, avg_interval, var_plm_rep,
           rarity_W, Wf1, bf1, Wf2, bf2, Wg1, bg1, Wg2, bg2,
           Wu, bu, Wr, br, Wc, bc):
    obsT = obs_emb.transpose(0, 1, 3, 2)
    avg_sm = avg_interval.transpose(1, 0, 2)
    maskT = (observed_mask.transpose(1, 2, 0)
             .reshape(_STEPS, _NODES, _BATCH // _BB, _BB)
             .transpose(2, 0, 1, 3))
    avgT = (avg_interval.transpose(1, 2, 0)
            .reshape(_STEPS, _NODES, _BATCH // _BB, _BB)
            .transpose(2, 0, 1, 3))
    def _wflat(w):
        wp = jnp.concatenate(
            [w[:, :_D], w[:, _D + 1:], w[:, _D:_D + 1],
             jnp.zeros((_QDIM, _FP - _NF, w.shape[2]), w.dtype)], axis=1)
        return wp.reshape(_QDIM * _FP, w.shape[2]).T

    WruT = _wflat(jnp.stack([Wr, Wu], axis=2).reshape(_QDIM, _NF, 2 * _D))
    WcT = _wflat(Wc)
    bruT = jnp.concatenate([br, bu], axis=1).T
    bcT = bc.T

    full = lambda nd: (lambda i: (0,) * nd)
    outT = pl.pallas_call(
        _rnn_body,
        grid=(_BATCH // _BB,),
        in_specs=[
            pl.BlockSpec((_BB, _STEPS, _D, _NODES), lambda i: (i, 0, 0, 0)),
            pl.BlockSpec((_BB, _STEPS, _NODES), lambda i: (i, 0, 0)),
            pl.BlockSpec((1, _STEPS, _NODES, _BB), lambda i: (i, 0, 0, 0)),
            pl.BlockSpec((_STEPS, _BB, _NODES), lambda i: (0, i, 0)),
            pl.BlockSpec((1, _STEPS, _NODES, _BB), lambda i: (i, 0, 0, 0)),
            pl.BlockSpec((_BB, _STEPS, _NODES), lambda i: (i, 0, 0)),
            pl.BlockSpec((_BB, 1), lambda i: (i, 0)),
            pl.BlockSpec((_PLM, _NODES), full(2)),
            pl.BlockSpec((_NODES, _NODES), full(2)),
            pl.BlockSpec((_H2, _PLM), full(2)),
            pl.BlockSpec((_H2, 1), full(2)),
            pl.BlockSpec((_QDIM, _H2), full(2)),
            pl.BlockSpec((_QDIM, 1), full(2)),
            pl.BlockSpec((_H2, _PLM), full(2)),
            pl.BlockSpec((_H2, 1), full(2)),
            pl.BlockSpec((8, _H2), full(2)),
            pl.BlockSpec((8, 1), full(2)),
            pl.BlockSpec((2 * _D, _QDIM * _FP), full(2)),
            pl.BlockSpec((_D, _QDIM * _FP), full(2)),
            pl.BlockSpec((2 * _D, _QDIM), full(2)),
            pl.BlockSpec((_D, _QDIM), full(2)),
        ],
        out_specs=pl.BlockSpec((_BB, _D, _NODES), lambda i: (i, 0, 0)),
        out_shape=jax.ShapeDtypeStruct((_BATCH, _D, _NODES), jnp.float32),
        scratch_shapes=[
            pltpu.VMEM((_NODES, _NODES), jnp.float32),
            pltpu.VMEM((_NODES, _NODES), jnp.float32),
            pltpu.VMEM((_QDIM * _FP, _NODES), jnp.bfloat16),
            pltpu.VMEM((2 * _D, _NODES), jnp.float32),
            pltpu.VMEM((_D, _NODES), jnp.float32),
            pltpu.VMEM((_STEPS, _NODES, _BB), jnp.float32),
            pltpu.VMEM((_STEPS, _BB, _NODES, _NODES), jnp.bfloat16),
            pltpu.VMEM((_STEPS, _BB, _NODES), jnp.float32),
        ],
        compiler_params=pltpu.CompilerParams(
            dimension_semantics=("arbitrary",)),
    )(obsT, observed_mask, maskT, avg_sm, avgT, avg_interval, lengths,
      var_plm_rep.T, rarity_W.T, Wf1.T, bf1.reshape(-1, 1),
      Wf2.T, bf2.reshape(-1, 1), Wg1.T, bg1.reshape(-1, 1),
      Wg2.T, bg2.reshape(-1, 1), WruT.astype(jnp.bfloat16),
      WcT.astype(jnp.bfloat16), bruT, bcT)
    return outT.transpose(0, 2, 1)
